# Initial kernel scaffold; baseline (speedup 1.0000x reference)
import jax
import jax.numpy as jnp
from jax.experimental import pallas as pl


def kernel(segment_embeddings, valid_mask, Wk, queries):
    raise NotImplementedError("write your pallas kernel here")


# speedup vs baseline: 1.0478x; 1.0478x over previous
import functools

import jax
import jax.numpy as jnp
from jax import lax
from jax.experimental import pallas as pl
from jax.experimental.pallas import tpu as pltpu

D_MODEL = 4096
RANK = 128
NUM_QUERIES = 8
BSZ = 4
N_SEG = 512
K_SEG = max(1, int(N_SEG * 0.3))
BLK = 256
GRID = (BSZ * N_SEG) // BLK
NEG = jnp.finfo(jnp.float32).min


def _body(emb_ref, mask_ref, wk_ref, q_ref, scores_ref, idx_ref,
          srow_ref):
    i = pl.program_id(0)
    x = emb_ref[...]
    wk = wk_ref[...]
    q = q_ref[...]

    keys = lax.dot_general(x, wk, (((1,), (1,)), ((), ())),
                           preferred_element_type=jnp.float32)
    sq_r = lax.dot_general(q, keys, (((1,), (1,)), ((), ())),
                           preferred_element_type=jnp.float32)
    s_row = jnp.max(sq_r, axis=0, keepdims=True)
    m = mask_ref[...]
    s_row = jnp.where(m > 0, s_row, NEG)
    scores_ref[...] = s_row

    b = i // (N_SEG // BLK)
    h = i % (N_SEG // BLK)
    srow_ref[pl.ds(b, 1), :, pl.ds(h * BLK, BLK)] = s_row.reshape(1, 1, BLK)

    @pl.when(i == GRID - 1)
    def _topk():
        for bb in range(BSZ):
            srow = srow_ref[bb, :, :]
            scol = jnp.transpose(srow)
            sj = jnp.broadcast_to(srow, (N_SEG, N_SEG))
            si = jnp.broadcast_to(scol, (N_SEG, N_SEG))
            ii = lax.broadcasted_iota(jnp.int32, (N_SEG, N_SEG), 0)
            jj = lax.broadcasted_iota(jnp.int32, (N_SEG, N_SEG), 1)
            beats = (sj > si) | ((sj == si) & (jj < ii))
            rank = jnp.sum(beats.astype(jnp.float32), axis=1,
                           keepdims=True)
            keep = rank < jnp.float32(K_SEG)
            keep_f = keep.astype(jnp.float32)
            tri = (jj <= ii).astype(jnp.float32)
            pos = lax.dot_general(tri, keep_f, (((1,), (0,)), ((), ())),
                                  preferred_element_type=jnp.float32)
            pos0 = pos - 1.0
            pp = lax.broadcasted_iota(jnp.int32, (N_SEG, K_SEG), 1)
            iic = lax.broadcasted_iota(jnp.int32, (N_SEG, K_SEG), 0)
            hit = (jnp.broadcast_to(pos0, (N_SEG, K_SEG))
                   == pp.astype(jnp.float32))
            hit = hit & jnp.broadcast_to(keep, (N_SEG, K_SEG))
            row = jnp.sum(jnp.where(hit, iic.astype(jnp.float32), 0.0),
                          axis=0, keepdims=True)
            idx_ref[bb:bb + 1, :] = row.astype(jnp.int32)


def kernel(segment_embeddings, valid_mask, Wk, queries):
    bsz, n_seg, d = segment_embeddings.shape
    emb2 = segment_embeddings.reshape(bsz * n_seg, d)
    maskf = valid_mask.reshape(1, bsz * n_seg).astype(jnp.float32)

    scores_flat, topk_idx = pl.pallas_call(
        _body,
        grid=(GRID,),
        in_specs=[
            pl.BlockSpec((BLK, D_MODEL), lambda i: (i, 0)),
            pl.BlockSpec((1, BLK), lambda i: (0, i)),
            pl.BlockSpec((RANK, D_MODEL), lambda i: (0, 0)),
            pl.BlockSpec((NUM_QUERIES, RANK), lambda i: (0, 0)),
        ],
        out_specs=[
            pl.BlockSpec((1, BLK), lambda i: (0, i)),
            pl.BlockSpec((BSZ, K_SEG), lambda i: (0, 0)),
        ],
        out_shape=[
            jax.ShapeDtypeStruct((1, bsz * n_seg), jnp.float32),
            jax.ShapeDtypeStruct((bsz, K_SEG), jnp.int32),
        ],
        scratch_shapes=[
            pltpu.VMEM((BSZ, 1, N_SEG), jnp.float32),
        ],
    )(emb2, maskf, Wk, queries)

    return scores_flat.reshape(bsz, n_seg), topk_idx


# speedup vs baseline: 1.1300x; 1.0784x over previous
import jax
import jax.numpy as jnp
from jax import lax
from jax.experimental import pallas as pl
from jax.experimental.pallas import tpu as pltpu

D_MODEL = 4096
RANK = 128
NUM_QUERIES = 8
BSZ = 4
N_SEG = 512
K_SEG = max(1, int(N_SEG * 0.3))
NEG = jnp.finfo(jnp.float32).min


def _body(emb_ref, mask_ref, wk_ref, q_ref, scores_ref, idx_ref):
    x = emb_ref[0]
    wk = wk_ref[...]
    q = q_ref[...]

    keys = lax.dot_general(x, wk, (((1,), (1,)), ((), ())),
                           preferred_element_type=jnp.float32)
    sq_r = lax.dot_general(q, keys, (((1,), (1,)), ((), ())),
                           preferred_element_type=jnp.float32)
    s_row = jnp.max(sq_r, axis=0, keepdims=True)
    m = mask_ref[0]
    s_row = jnp.where(m > 0, s_row, NEG)
    scores_ref[0] = s_row

    scol = jnp.transpose(s_row)
    sj = jnp.broadcast_to(s_row, (N_SEG, N_SEG))
    si = jnp.broadcast_to(scol, (N_SEG, N_SEG))
    ii = lax.broadcasted_iota(jnp.int32, (N_SEG, N_SEG), 0)
    jj = lax.broadcasted_iota(jnp.int32, (N_SEG, N_SEG), 1)
    beats = (sj > si) | ((sj == si) & (jj < ii))
    rank = jnp.sum(beats.astype(jnp.float32), axis=1, keepdims=True)
    keep = rank < jnp.float32(K_SEG)
    keep_f = keep.astype(jnp.float32)
    tri = (jj <= ii).astype(jnp.float32)
    pos = lax.dot_general(tri, keep_f, (((1,), (0,)), ((), ())),
                          preferred_element_type=jnp.float32)
    pos0 = pos - 1.0
    pp = lax.broadcasted_iota(jnp.int32, (N_SEG, K_SEG), 1)
    iic = lax.broadcasted_iota(jnp.int32, (N_SEG, K_SEG), 0)
    hit = (jnp.broadcast_to(pos0, (N_SEG, K_SEG)) == pp.astype(jnp.float32))
    hit = hit & jnp.broadcast_to(keep, (N_SEG, K_SEG))
    row = jnp.sum(jnp.where(hit, iic.astype(jnp.float32), 0.0),
                  axis=0, keepdims=True)
    idx_ref[0] = row.astype(jnp.int32)


def kernel(segment_embeddings, valid_mask, Wk, queries):
    bsz, n_seg, d = segment_embeddings.shape
    maskf = valid_mask.reshape(bsz, 1, n_seg).astype(jnp.float32)

    scores, topk_idx = pl.pallas_call(
        _body,
        grid=(bsz,),
        in_specs=[
            pl.BlockSpec((1, N_SEG, D_MODEL), lambda b: (b, 0, 0)),
            pl.BlockSpec((1, 1, N_SEG), lambda b: (b, 0, 0)),
            pl.BlockSpec((RANK, D_MODEL), lambda b: (0, 0)),
            pl.BlockSpec((NUM_QUERIES, RANK), lambda b: (0, 0)),
        ],
        out_specs=[
            pl.BlockSpec((1, 1, N_SEG), lambda b: (b, 0, 0)),
            pl.BlockSpec((1, 1, K_SEG), lambda b: (b, 0, 0)),
        ],
        out_shape=[
            jax.ShapeDtypeStruct((bsz, 1, n_seg), jnp.float32),
            jax.ShapeDtypeStruct((bsz, 1, K_SEG), jnp.int32),
        ],
    )(segment_embeddings, maskf, Wk, queries)

    return scores.reshape(bsz, n_seg), topk_idx.reshape(bsz, K_SEG)
